# baseline (speedup 1.0000x reference)
import functools

import jax
import jax.numpy as jnp
from jax import lax
from jax.experimental import pallas as pl
from jax.experimental.pallas import tpu as pltpu
from jax.experimental.pallas import tpu_sc as plsc

_B, _T = 16, 4096
_L = 16
_NBLK = _T // _L
_CH = 128
_NCH = _T // _CH
_TPW = 4


def _body(pred_hbm, out_hbm, len_hbm, stage_in, stage_out, len_sh,
          in_v, out_v, len_v, diag_v, sem_in, sem_row, sem_out):
    w = lax.axis_index("s")

    loads = []
    for j in range(_TPW):
        t = w * _TPW + j
        br = t // _NCH
        kk = t % _NCH
        loads.append(pltpu.async_copy(
            pred_hbm.at[
                pl.ds(pl.multiple_of(br * 8, 8), 8),
                pl.ds(pl.multiple_of(kk * _CH, _CH), _CH),
            ],
            stage_in.at[br, kk],
            sem_in,
        ))
    for cp in loads:
        cp.wait()
    plsc.subcore_barrier()

    in_v[pl.ds(0, _L)] = jnp.zeros((_L,), jnp.int32)
    rbr = w // 8
    rr = w % 8
    rows = [
        pltpu.async_copy(
            stage_in.at[rbr, k, rr],
            in_v.at[pl.ds(_L + k * _CH, _CH)],
            sem_row,
        )
        for k in range(_NCH)
    ]
    for cp in rows:
        cp.wait()

    iota = lax.iota(jnp.int32, _L)
    iota1 = iota + jnp.ones((_L,), jnp.int32)
    zero_v = jnp.zeros((_L,), jnp.int32)
    one_v = jnp.ones((_L,), jnp.int32)
    shift_idx = [jnp.maximum(iota - k, 0) for k in (1, 2, 4, 8)]
    shift_keep = [iota >= k for k in (1, 2, 4, 8)]

    gdn = lax.GatherDimensionNumbers(
        offset_dims=(), collapsed_slice_dims=(0,), start_index_map=(0,)
    )

    def lane_gather(v, idx):
        return lax.gather(
            v, idx[:, None], gdn, slice_sizes=(1,),
            mode=lax.GatherScatterMode.PROMISE_IN_BOUNDS,
        )

    def prefix16(v):
        for idx, keep in zip(shift_idx, shift_keep):
            v = v + jnp.where(keep, lane_gather(v, idx), zero_v)
        return v

    def step(i, cnt_vec):
        x = in_v[pl.ds(_L + i * _L, _L)]
        xp = in_v[pl.ds(_L - 1 + i * _L, _L)]
        m = (x != xp) & (x != zero_v)
        mi = jnp.where(m, one_v, zero_v)
        cum = prefix16(mi)
        pc = plsc.all_reduce_population_count(m)
        kept_dest = cnt_vec + cum - one_v
        top = jnp.full((_L,), _T - i * _L, jnp.int32)
        drop_dest = top + cnt_vec - iota1 + cum
        dest = jnp.where(m, kept_dest, drop_dest)
        val = jnp.where(m, x, zero_v)
        plsc.store_scatter(out_v, [dest], val)
        return cnt_vec + pc

    cnt_vec = lax.fori_loop(0, _NBLK, step, jnp.zeros((_L,), jnp.int32))

    rows_out = [
        pltpu.async_copy(
            out_v.at[pl.ds(k * _CH, _CH)],
            stage_out.at[rbr, k, rr],
            sem_row,
        )
        for k in range(_NCH)
    ]
    len_v[...] = cnt_vec
    pltpu.sync_copy(len_v, len_sh.at[w])
    for cp in rows_out:
        cp.wait()
    plsc.subcore_barrier()

    stores = []
    for j in range(_TPW):
        t = w * _TPW + j
        br = t // _NCH
        kk = t % _NCH
        stores.append(pltpu.async_copy(
            stage_out.at[br, kk],
            out_hbm.at[
                pl.ds(pl.multiple_of(br * 8, 8), 8),
                pl.ds(pl.multiple_of(kk * _CH, _CH), _CH),
            ],
            sem_out,
        ))

    @pl.when(w == 0)
    def _():
        pltpu.sync_copy(len_sh, diag_v)
        lens = plsc.load_gather(diag_v, [iota, iota])
        len_v[...] = lens
        pltpu.sync_copy(len_v, len_hbm)

    for cp in stores:
        cp.wait()


@jax.jit
def _run(predictions):
    mesh = plsc.VectorSubcoreMesh(
        core_axis_name="c", subcore_axis_name="s", num_cores=1, num_subcores=16
    )
    k = pl.kernel(
        _body,
        out_type=[
            jax.ShapeDtypeStruct((_B, _T), jnp.int32),
            jax.ShapeDtypeStruct((_B,), jnp.int32),
        ],
        mesh=mesh,
        scratch_types=[
            pltpu.VMEM_SHARED((2, _NCH, 8, _CH), jnp.int32),
            pltpu.VMEM_SHARED((2, _NCH, 8, _CH), jnp.int32),
            pltpu.VMEM_SHARED((_B, _L), jnp.int32),
            pltpu.VMEM((_L + _T,), jnp.int32),
            pltpu.VMEM((_T,), jnp.int32),
            pltpu.VMEM((_L,), jnp.int32),
            pltpu.VMEM((_B, _L), jnp.int32),
            pltpu.SemaphoreType.DMA,
            pltpu.SemaphoreType.DMA,
            pltpu.SemaphoreType.DMA,
        ],
        compiler_params=pltpu.CompilerParams(
            needs_layout_passes=False,
            use_tc_tiling_on_sc=True,
            disable_bounds_checks=True,
            disable_semaphore_checks=True,
            skip_device_barrier=True,
        ),
    )
    compact, lens = k(predictions)
    return compact, lens


def kernel(predictions):
    return _run(predictions)


# speedup vs baseline: 1.0558x; 1.0558x over previous
import functools

import jax
import jax.numpy as jnp
from jax import lax
from jax.experimental import pallas as pl
from jax.experimental.pallas import tpu as pltpu
from jax.experimental.pallas import tpu_sc as plsc

_B, _T = 16, 4096
_L = 16
_NBLK = _T // _L
_CH = 128
_NCH = _T // _CH
_TPW = 4


def _body(pred_hbm, out_hbm, len_hbm, stage_in, stage_out, len_sh,
          in_v, out_v, len_v, diag_v, sem_in, sem_row, sem_out):
    w = lax.axis_index("s")

    loads = []
    for j in range(_TPW):
        t = w * _TPW + j
        br = t // _NCH
        kk = t % _NCH
        loads.append(pltpu.async_copy(
            pred_hbm.at[
                pl.ds(pl.multiple_of(br * 8, 8), 8),
                pl.ds(pl.multiple_of(kk * _CH, _CH), _CH),
            ],
            stage_in.at[br, kk],
            sem_in,
        ))
    in_v[pl.ds(0, _L)] = jnp.zeros((_L,), jnp.int32)

    for cp in loads:
        cp.wait()
    plsc.subcore_barrier()

    rbr = w // 8
    rr = w % 8
    rows = [
        pltpu.async_copy(
            stage_in.at[rbr, k, rr],
            in_v.at[pl.ds(_L + k * _CH, _CH)],
            sem_row,
        )
        for k in range(_NCH)
    ]
    for cp in rows:
        cp.wait()

    iota = lax.iota(jnp.int32, _L)
    iota1 = iota + jnp.ones((_L,), jnp.int32)
    zero_v = jnp.zeros((_L,), jnp.int32)
    one_v = jnp.ones((_L,), jnp.int32)

    def step(i, cnt_vec):
        x = in_v[pl.ds(_L + i * _L, _L)]
        xp = in_v[pl.ds(_L - 1 + i * _L, _L)]
        m = (x != xp) & (x != zero_v)
        mi = jnp.where(m, one_v, zero_v)
        cum = plsc.cumsum(mi)
        pc = plsc.all_reduce_population_count(m)
        kept_dest = cnt_vec + cum - one_v
        top = jnp.full((_L,), _T - i * _L, jnp.int32)
        drop_dest = top + cnt_vec - iota1 + cum
        dest = jnp.where(m, kept_dest, drop_dest)
        val = jnp.where(m, x, zero_v)
        plsc.store_scatter(out_v, [dest], val)
        return cnt_vec + pc

    cnt_vec = lax.fori_loop(0, _NBLK, step, jnp.zeros((_L,), jnp.int32))

    rows_out = [
        pltpu.async_copy(
            out_v.at[pl.ds(k * _CH, _CH)],
            stage_out.at[rbr, k, rr],
            sem_row,
        )
        for k in range(_NCH)
    ]
    len_v[...] = cnt_vec
    pltpu.sync_copy(len_v, len_sh.at[w])
    for cp in rows_out:
        cp.wait()
    plsc.subcore_barrier()

    stores = []
    for j in range(_TPW):
        t = w * _TPW + j
        br = t // _NCH
        kk = t % _NCH
        stores.append(pltpu.async_copy(
            stage_out.at[br, kk],
            out_hbm.at[
                pl.ds(pl.multiple_of(br * 8, 8), 8),
                pl.ds(pl.multiple_of(kk * _CH, _CH), _CH),
            ],
            sem_out,
        ))

    @pl.when(w == 0)
    def _():
        pltpu.sync_copy(len_sh, diag_v)
        lens = plsc.load_gather(diag_v, [iota, iota])
        len_v[...] = lens
        pltpu.sync_copy(len_v, len_hbm)

    for cp in stores:
        cp.wait()


@jax.jit
def _run(predictions):
    mesh = plsc.VectorSubcoreMesh(
        core_axis_name="c", subcore_axis_name="s", num_cores=1, num_subcores=16
    )
    k = pl.kernel(
        _body,
        out_type=[
            jax.ShapeDtypeStruct((_B, _T), jnp.int32),
            jax.ShapeDtypeStruct((_B,), jnp.int32),
        ],
        mesh=mesh,
        scratch_types=[
            pltpu.VMEM_SHARED((2, _NCH, 8, _CH), jnp.int32),
            pltpu.VMEM_SHARED((2, _NCH, 8, _CH), jnp.int32),
            pltpu.VMEM_SHARED((_B, _L), jnp.int32),
            pltpu.VMEM((_L + _T,), jnp.int32),
            pltpu.VMEM((_T,), jnp.int32),
            pltpu.VMEM((_L,), jnp.int32),
            pltpu.VMEM((_B, _L), jnp.int32),
            pltpu.SemaphoreType.DMA,
            pltpu.SemaphoreType.DMA,
            pltpu.SemaphoreType.DMA,
        ],
        compiler_params=pltpu.CompilerParams(
            needs_layout_passes=False,
            use_tc_tiling_on_sc=True,
            disable_bounds_checks=True,
            disable_semaphore_checks=True,
            skip_device_barrier=True,
        ),
    )
    compact, lens = k(predictions)
    return compact, lens


def kernel(predictions):
    return _run(predictions)
